# Initial kernel scaffold; baseline (speedup 1.0000x reference)
import jax
import jax.numpy as jnp
from jax.experimental import pallas as pl


def kernel(states, actions, returns_to_go, time_steps, timestep_table, Ws, bs, Wa, ba, Wr, br):
    raise NotImplementedError("write your pallas kernel here")


# speedup vs baseline: 1.2098x; 1.2098x over previous
import functools

import jax
import jax.numpy as jnp
from jax import lax
from jax.experimental import pallas as pl
from jax.experimental.pallas import tpu as pltpu
from jax.experimental.pallas import tpu_sc as plsc

_NC, _NS = 2, 16
_NW = _NC * _NS
_CH = 128


def _sc_gather(idx, table, rows, d):
    rpw = rows // _NW
    nchunk = rpw // _CH
    mesh = plsc.VectorSubcoreMesh(core_axis_name="c", subcore_axis_name="s")

    @functools.partial(
        pl.kernel,
        out_type=jax.ShapeDtypeStruct((rows, d), jnp.float32),
        mesh=mesh,
        scratch_types=[
            pltpu.VMEM((_CH,), jnp.int32),
            pltpu.VMEM((_CH, d), jnp.float32),
            pltpu.SemaphoreType.DMA,
        ],
    )
    def gather_kernel(idx_hbm, table_hbm, out_hbm, idx_v, rows_v, sem):
        wid = lax.axis_index("s") * _NC + lax.axis_index("c")
        base = wid * rpw

        @pl.loop(0, nchunk)
        def _(j):
            off = base + j * _CH
            pltpu.sync_copy(idx_hbm.at[pl.ds(off, _CH)], idx_v)
            pltpu.async_copy(table_hbm.at[idx_v], rows_v, sem).wait()
            pltpu.sync_copy(rows_v, out_hbm.at[pl.ds(off, _CH)])

    return gather_kernel(idx, table)


def _tc_body(s_ref, a_ref, r_ref, t_ref, ws_ref, bs_ref, wa_ref, ba_ref,
             wr_ref, br_ref, o_ref):
    t = t_ref[...]
    se = jnp.dot(s_ref[...], ws_ref[...],
                 preferred_element_type=jnp.float32) + bs_ref[...][None, :] + t
    ae = jnp.dot(a_ref[...], wa_ref[...],
                 preferred_element_type=jnp.float32) + ba_ref[...][None, :] + t
    re = r_ref[...] * wr_ref[...] + br_ref[...][None, :] + t
    o_ref[:, 0, :] = re
    o_ref[:, 1, :] = se
    o_ref[:, 2, :] = ae


def kernel(states, actions, returns_to_go, time_steps, timestep_table,
           Ws, bs, Wa, ba, Wr, br):
    b, l, sd = states.shape
    ad = actions.shape[-1]
    d = timestep_table.shape[-1]
    rows = b * l

    idx = time_steps.reshape(rows).astype(jnp.int32)
    time_emb = _sc_gather(idx, timestep_table, rows, d)

    rb = 2048
    grid = rows // rb
    out = pl.pallas_call(
        _tc_body,
        grid=(grid,),
        in_specs=[
            pl.BlockSpec((rb, sd), lambda i: (i, 0)),
            pl.BlockSpec((rb, ad), lambda i: (i, 0)),
            pl.BlockSpec((rb, 1), lambda i: (i, 0)),
            pl.BlockSpec((rb, d), lambda i: (i, 0)),
            pl.BlockSpec((sd, d), lambda i: (0, 0)),
            pl.BlockSpec((d,), lambda i: (0,)),
            pl.BlockSpec((ad, d), lambda i: (0, 0)),
            pl.BlockSpec((d,), lambda i: (0,)),
            pl.BlockSpec((1, d), lambda i: (0, 0)),
            pl.BlockSpec((d,), lambda i: (0,)),
        ],
        out_specs=pl.BlockSpec((rb, 3, d), lambda i: (i, 0, 0)),
        out_shape=jax.ShapeDtypeStruct((rows, 3, d), jnp.float32),
    )(states.reshape(rows, sd), actions.reshape(rows, ad),
      returns_to_go.reshape(rows, 1), time_emb, Ws, bs, Wa, ba, Wr, br)
    return out.reshape(b, 3 * l, d)


# speedup vs baseline: 1.9432x; 1.6062x over previous
import functools

import jax
import jax.numpy as jnp
from jax import lax
from jax.experimental import pallas as pl
from jax.experimental.pallas import tpu as pltpu
from jax.experimental.pallas import tpu_sc as plsc

_NC, _NS = 2, 16
_NW = _NC * _NS
_CH = 128


def _sc_gather(idx, table, rows, d):
    rpw = rows // _NW
    nchunk = rpw // _CH
    mesh = plsc.VectorSubcoreMesh(core_axis_name="c", subcore_axis_name="s")

    @functools.partial(
        pl.kernel,
        out_type=jax.ShapeDtypeStruct((rows, d), jnp.float32),
        mesh=mesh,
        scratch_types=[
            pltpu.VMEM((_CH,), jnp.int32),
            pltpu.VMEM((_CH, d), jnp.float32),
            pltpu.SemaphoreType.DMA,
        ],
    )
    def gather_kernel(idx_hbm, table_hbm, out_hbm, idx_v, rows_v, sem):
        wid = lax.axis_index("s") * _NC + lax.axis_index("c")
        base = wid * rpw

        @pl.loop(0, nchunk)
        def _(j):
            off = base + j * _CH
            pltpu.sync_copy(idx_hbm.at[pl.ds(off, _CH)], idx_v)
            pltpu.async_copy(table_hbm.at[idx_v], rows_v, sem).wait()
            pltpu.sync_copy(rows_v, out_hbm.at[pl.ds(off, _CH)])

    return gather_kernel(idx, table)


def _make_tc_body(bb, l, d):
    def _tc_body(s_ref, a_ref, r_ref, t_ref, ws_ref, bs_ref, wa_ref, ba_ref,
                 wr_ref, br_ref, o_ref, scratch_ref):
        t = t_ref[...]
        se = jnp.dot(s_ref[...], ws_ref[...],
                     preferred_element_type=jnp.float32) + bs_ref[...][None, :] + t
        ae = jnp.dot(a_ref[...], wa_ref[...],
                     preferred_element_type=jnp.float32) + ba_ref[...][None, :] + t
        re = r_ref[...] * wr_ref[...] + br_ref[...][None, :] + t
        sv = scratch_ref.reshape(bb * l, 3, d)
        sv[:, 0, :] = re
        sv[:, 1, :] = se
        sv[:, 2, :] = ae
        o_ref[...] = scratch_ref.reshape(bb, 3 * l, d)[...]
    return _tc_body


def kernel(states, actions, returns_to_go, time_steps, timestep_table,
           Ws, bs, Wa, ba, Wr, br):
    b, l, sd = states.shape
    ad = actions.shape[-1]
    d = timestep_table.shape[-1]
    rows = b * l

    idx = time_steps.reshape(rows).astype(jnp.int32)
    time_emb = _sc_gather(idx, timestep_table, rows, d)

    bb = 32
    rb = bb * l
    grid = b // bb
    out = pl.pallas_call(
        _make_tc_body(bb, l, d),
        grid=(grid,),
        in_specs=[
            pl.BlockSpec((rb, sd), lambda i: (i, 0)),
            pl.BlockSpec((rb, ad), lambda i: (i, 0)),
            pl.BlockSpec((rb, 1), lambda i: (i, 0)),
            pl.BlockSpec((rb, d), lambda i: (i, 0)),
            pl.BlockSpec((sd, d), lambda i: (0, 0)),
            pl.BlockSpec((d,), lambda i: (0,)),
            pl.BlockSpec((ad, d), lambda i: (0, 0)),
            pl.BlockSpec((d,), lambda i: (0,)),
            pl.BlockSpec((1, d), lambda i: (0, 0)),
            pl.BlockSpec((d,), lambda i: (0,)),
        ],
        out_specs=pl.BlockSpec((bb, 3 * l, d), lambda i: (i, 0, 0)),
        out_shape=jax.ShapeDtypeStruct((b, 3 * l, d), jnp.float32),
        scratch_shapes=[pltpu.VMEM((3 * rb, d), jnp.float32)],
    )(states.reshape(rows, sd), actions.reshape(rows, ad),
      returns_to_go.reshape(rows, 1), time_emb, Ws, bs, Wa, ba, Wr, br)
    return out
